# baseline (speedup 1.0000x reference)
import jax
import jax.numpy as jnp
from jax import lax
from jax.experimental import pallas as pl
from jax.experimental.pallas import tpu as pltpu

B = 16384
UNI = 128
HID = 64
HEADS = 4
C1 = HID // HEADS
BT = 1024
BT2 = BT // 2

_TRANS_RHS = (((1,), (1,)), ((), ()))
_TRANS_LHS = (((0,), (1,)), ((), ()))


def _leaky(x):
    return jnp.maximum(x, 0.2 * x)


def _elu(x):
    return jnp.where(x > 0, x, jnp.exp(x) - 1.0)


def _gnn_kernel(t_ref, a_ref, v_ref,
                wt_ref, wa_ref, wv_ref,
                wl1_ref, wr1_ref, att1_ref,
                wl2_ref, wr2_ref, att2_ref,
                bvec_ref, out_ref,
                ql_ref, qr_ref, cl_ref, cr_ref,
                w2l_ref, w2r_ref, ma1_ref, ma2_ref, b2v_ref):
    f32 = jnp.float32

    def dg(x, y, dims):
        return lax.dot_general(x, y, dims, preferred_element_type=f32)

    @pl.when(pl.program_id(0) == 0)
    def _prep():
        bv = bvec_ref[...]
        bn = [bv[i:i + 1] for i in range(3)]
        bl1, br1 = bv[3:4], bv[4:5]
        bl2, br2 = bv[6:7], bv[7:8]
        wl1 = wl1_ref[...]
        wr1 = wr1_ref[...]
        wn = (wt_ref[...], wa_ref[...], wv_ref[...])
        z128 = jnp.zeros((UNI, HID), f32)
        z64 = jnp.zeros((HID, HID), f32)
        for i in range(3):
            ql = dg(wn[i], wl1, _TRANS_LHS)
            qr = dg(wn[i], wr1, _TRANS_LHS)
            ql_ref[i, :UNI, :HID] = ql
            ql_ref[i, :UNI, HID:] = z128
            ql_ref[i, UNI:, :HID] = z128
            ql_ref[i, UNI:, HID:] = ql
            qr_ref[i, :UNI, :HID] = qr
            qr_ref[i, :UNI, HID:] = z128
            qr_ref[i, UNI:, :HID] = z128
            qr_ref[i, UNI:, HID:] = qr
            cl = dg(bn[i], wl1, _TRANS_RHS) + bl1
            cr = dg(bn[i], wr1, _TRANS_RHS) + br1
            cl_ref[i:i + 1, :HID] = cl
            cl_ref[i:i + 1, HID:] = cl
            cr_ref[i:i + 1, :HID] = cr
            cr_ref[i:i + 1, HID:] = cr
        rh = lax.broadcasted_iota(jnp.int32, (HID, HID), 0)
        ch = lax.broadcasted_iota(jnp.int32, (HID, HID), 1)
        ident = (rh == ch).astype(f32)
        w2lt = dg(ident, wl2_ref[...], _TRANS_RHS)
        w2rt = dg(ident, wr2_ref[...], _TRANS_RHS)
        w2l_ref[:HID, :HID] = w2lt
        w2l_ref[:HID, HID:] = z64
        w2l_ref[HID:, :HID] = z64
        w2l_ref[HID:, HID:] = w2lt
        w2r_ref[:HID, :HID] = w2rt
        w2r_ref[:HID, HID:] = z64
        w2r_ref[HID:, :HID] = z64
        w2r_ref[HID:, HID:] = w2rt
        att1_col = dg(ident, att1_ref[...], _TRANS_RHS)
        ma1 = att1_col * (rh // C1 == ch // C1).astype(f32)
        ma1_ref[:HID, :HID] = ma1
        ma1_ref[:HID, HID:] = z64
        ma1_ref[HID:, :HID] = z64
        ma1_ref[HID:, HID:] = ma1
        att2_col = dg(ident, att2_ref[...], _TRANS_RHS)
        a2 = jnp.broadcast_to(att2_col, (HID, HID))
        ma2_ref[:HID, :HID] = a2
        ma2_ref[:HID, HID:] = z64
        ma2_ref[HID:, :HID] = z64
        ma2_ref[HID:, HID:] = a2
        bias1, bias2 = bv[5:6], bv[8:9]
        b2v_ref[0:1, :HID] = bias1
        b2v_ref[0:1, HID:] = bias1
        b2v_ref[1:2, :HID] = bl2
        b2v_ref[1:2, HID:] = bl2
        b2v_ref[2:3, :HID] = br2
        b2v_ref[2:3, HID:] = br2
        b2v_ref[3:4, :HID] = bias2
        b2v_ref[3:4, HID:] = bias2

    feats = (t_ref[...], a_ref[...], v_ref[...])
    b2v = b2v_ref[...]
    bias1, bl2, br2, bias2 = (b2v[i:i + 1] for i in range(4))
    cl = cl_ref[...]
    cr = cr_ref[...]

    xl = [jnp.dot(feats[i], ql_ref[i], preferred_element_type=f32) + cl[i:i + 1]
          for i in range(3)]
    xr = [jnp.dot(feats[i], qr_ref[i], preferred_element_type=f32) + cr[i:i + 1]
          for i in range(3)]

    def gat(xls, xrs, logit_ref):
        outs = []
        for d in range(3):
            a, b = [s for s in range(3) if s != d]
            ea = _leaky(xls[a] + xrs[d])
            eb = _leaky(xls[b] + xrs[d])
            dlog = jnp.dot(eb - ea, logit_ref[...],
                           preferred_element_type=f32)
            sa = 1.0 / (1.0 + jnp.exp(dlog))
            outs.append(xls[b] + sa * (xls[a] - xls[b]))
        return outs

    h = [_elu(o + bias1) for o in gat(xl, xr, ma1_ref)]

    yl = [jnp.dot(h[i], w2l_ref[...], preferred_element_type=f32) + bl2
          for i in range(3)]
    yr = [jnp.dot(h[i], w2r_ref[...], preferred_element_type=f32) + br2
          for i in range(3)]
    o2 = gat(yl, yr, ma2_ref)
    out_ref[...] = (o2[0] + o2[1] + o2[2]) * (1.0 / 3.0) + bias2


@jax.jit
def kernel(text_features, audio_features, video_features, W_text, b_text,
           W_audio, b_audio, W_video, b_video, Wl1, bl1, Wr1, br1, att1,
           bias1, Wl2, bl2, Wr2, br2, att2, bias2):
    f32 = jnp.float32
    bvec = jnp.stack([b_text, b_audio, b_video, bl1, br1, bias1,
                      bl2, br2, bias2]).astype(f32)
    att1_row = att1.reshape(1, HEADS * C1).astype(f32)

    tp = text_features.reshape(B // 2, 2 * UNI)
    ap = audio_features.reshape(B // 2, 2 * UNI)
    vp = video_features.reshape(B // 2, 2 * UNI)

    grid = (B // BT,)
    data_spec = pl.BlockSpec((BT2, 2 * UNI), lambda i: (i, 0))
    w_proj = pl.BlockSpec((HID, UNI), lambda i: (0, 0))
    w_hid = pl.BlockSpec((HID, HID), lambda i: (0, 0))
    vec = pl.BlockSpec((1, HID), lambda i: (0, 0))

    out = pl.pallas_call(
        _gnn_kernel,
        grid=grid,
        in_specs=[
            data_spec, data_spec, data_spec,
            w_proj, w_proj, w_proj,
            w_hid, w_hid, vec,
            w_hid, w_hid, vec,
            pl.BlockSpec((9, HID), lambda i: (0, 0)),
        ],
        out_specs=pl.BlockSpec((BT2, 2 * HID), lambda i: (i, 0)),
        out_shape=jax.ShapeDtypeStruct((B // 2, 2 * HID), f32),
        scratch_shapes=[
            pltpu.VMEM((3, 2 * UNI, 2 * HID), f32),
            pltpu.VMEM((3, 2 * UNI, 2 * HID), f32),
            pltpu.VMEM((3, 2 * HID), f32),
            pltpu.VMEM((3, 2 * HID), f32),
            pltpu.VMEM((2 * HID, 2 * HID), f32),
            pltpu.VMEM((2 * HID, 2 * HID), f32),
            pltpu.VMEM((2 * HID, 2 * HID), f32),
            pltpu.VMEM((2 * HID, 2 * HID), f32),
            pltpu.VMEM((4, 2 * HID), f32),
        ],
    )(
        tp, ap, vp,
        W_text, W_audio, W_video,
        Wl1, Wr1, att1_row,
        Wl2, Wr2, att2,
        bvec,
    )
    return out.reshape(B, HID)


# speedup vs baseline: 1.8000x; 1.8000x over previous
import jax
import jax.numpy as jnp
from jax import lax
from jax.experimental import pallas as pl
from jax.experimental.pallas import tpu as pltpu

B = 16384
UNI = 128
HID = 64
HEADS = 4
C1 = HID // HEADS
BT2 = 512
NG = (B // 2) // BT2

_TRANS_RHS = (((1,), (1,)), ((), ()))
_TRANS_LHS = (((0,), (1,)), ((), ()))


def _leaky(x):
    return jnp.maximum(x, 0.2 * x)


def _elu(x):
    return jnp.where(x > 0, x, jnp.exp(x) - 1.0)


def _gnn_kernel(ta_ref, tb_ref, aa_ref, ab_ref, va_ref, vb_ref,
                wt_ref, wa_ref, wv_ref,
                wl1_ref, wr1_ref, att1_ref,
                wl2_ref, wr2_ref, att2_ref,
                bvec_ref, out_ref,
                ql_ref, qr_ref, cl_ref, cr_ref,
                w2l_ref, w2r_ref, ma1_ref, ma2_ref, b2v_ref):
    f32 = jnp.float32

    def dg(x, y, dims):
        return lax.dot_general(x, y, dims, preferred_element_type=f32)

    @pl.when(pl.program_id(0) == 0)
    def _prep():
        bv = bvec_ref[...]
        bn = [bv[i:i + 1] for i in range(3)]
        bl1, br1 = bv[3:4], bv[4:5]
        bl2, br2 = bv[6:7], bv[7:8]
        wl1 = wl1_ref[...]
        wr1 = wr1_ref[...]
        wn = (wt_ref[...], wa_ref[...], wv_ref[...])
        z128 = jnp.zeros((UNI, HID), f32)
        z64 = jnp.zeros((HID, HID), f32)
        for i in range(3):
            ql = dg(wn[i], wl1, _TRANS_LHS)
            qr = dg(wn[i], wr1, _TRANS_LHS)
            ql_ref[i, 0, :, :HID] = ql
            ql_ref[i, 0, :, HID:] = z128
            ql_ref[i, 1, :, :HID] = z128
            ql_ref[i, 1, :, HID:] = ql
            qr_ref[i, 0, :, :HID] = qr
            qr_ref[i, 0, :, HID:] = z128
            qr_ref[i, 1, :, :HID] = z128
            qr_ref[i, 1, :, HID:] = qr
            cl = dg(bn[i], wl1, _TRANS_RHS) + bl1
            cr = dg(bn[i], wr1, _TRANS_RHS) + br1
            cl_ref[i:i + 1, :HID] = cl
            cl_ref[i:i + 1, HID:] = cl
            cr_ref[i:i + 1, :HID] = cr
            cr_ref[i:i + 1, HID:] = cr
        rh = lax.broadcasted_iota(jnp.int32, (HID, HID), 0)
        ch = lax.broadcasted_iota(jnp.int32, (HID, HID), 1)
        ident = (rh == ch).astype(f32)
        w2lt = dg(ident, wl2_ref[...], _TRANS_RHS)
        w2rt = dg(ident, wr2_ref[...], _TRANS_RHS)
        w2l_ref[:HID, :HID] = w2lt
        w2l_ref[:HID, HID:] = z64
        w2l_ref[HID:, :HID] = z64
        w2l_ref[HID:, HID:] = w2lt
        w2r_ref[:HID, :HID] = w2rt
        w2r_ref[:HID, HID:] = z64
        w2r_ref[HID:, :HID] = z64
        w2r_ref[HID:, HID:] = w2rt
        att1_col = dg(ident, att1_ref[...], _TRANS_RHS)
        ma1 = att1_col * (rh // C1 == ch // C1).astype(f32)
        ma1_ref[:HID, :HID] = ma1
        ma1_ref[:HID, HID:] = z64
        ma1_ref[HID:, :HID] = z64
        ma1_ref[HID:, HID:] = ma1
        att2_col = dg(ident, att2_ref[...], _TRANS_RHS)
        a2 = jnp.broadcast_to(att2_col, (HID, HID))
        ma2_ref[:HID, :HID] = a2
        ma2_ref[:HID, HID:] = z64
        ma2_ref[HID:, :HID] = z64
        ma2_ref[HID:, HID:] = a2
        bias1, bias2 = bv[5:6], bv[8:9]
        b2v_ref[0:1, :HID] = bias1
        b2v_ref[0:1, HID:] = bias1
        b2v_ref[1:2, :HID] = bl2
        b2v_ref[1:2, HID:] = bl2
        b2v_ref[2:3, :HID] = br2
        b2v_ref[2:3, HID:] = br2
        b2v_ref[3:4, :HID] = bias2
        b2v_ref[3:4, HID:] = bias2

    fa = (ta_ref[...], aa_ref[...], va_ref[...])
    fb = (tb_ref[...], ab_ref[...], vb_ref[...])
    b2v = b2v_ref[...]
    bias1, bl2, br2, bias2 = (b2v[i:i + 1] for i in range(4))
    cl = cl_ref[...]
    cr = cr_ref[...]

    def mm(x, w):
        return jnp.dot(x, w, preferred_element_type=f32)

    xl = [mm(fa[i], ql_ref[i, 0]) + mm(fb[i], ql_ref[i, 1]) + cl[i:i + 1]
          for i in range(3)]
    xr = [mm(fa[i], qr_ref[i, 0]) + mm(fb[i], qr_ref[i, 1]) + cr[i:i + 1]
          for i in range(3)]

    def gat(xls, xrs, logit_ref):
        outs = []
        for d in range(3):
            a, b = [s for s in range(3) if s != d]
            ea = _leaky(xls[a] + xrs[d])
            eb = _leaky(xls[b] + xrs[d])
            dlog = mm(eb - ea, logit_ref[...])
            sa = 1.0 / (1.0 + jnp.exp(dlog))
            outs.append(xls[b] + sa * (xls[a] - xls[b]))
        return outs

    h = [_elu(o + bias1) for o in gat(xl, xr, ma1_ref)]

    yl = [mm(h[i], w2l_ref[...]) + bl2 for i in range(3)]
    yr = [mm(h[i], w2r_ref[...]) + br2 for i in range(3)]
    o2 = gat(yl, yr, ma2_ref)
    res = (o2[0] + o2[1] + o2[2]) * (1.0 / 3.0) + bias2
    out_ref[0] = res[:, :HID]
    out_ref[1] = res[:, HID:]


@jax.jit
def kernel(text_features, audio_features, video_features, W_text, b_text,
           W_audio, b_audio, W_video, b_video, Wl1, bl1, Wr1, br1, att1,
           bias1, Wl2, bl2, Wr2, br2, att2, bias2):
    f32 = jnp.float32
    bvec = jnp.stack([b_text, b_audio, b_video, bl1, br1, bias1,
                      bl2, br2, bias2]).astype(f32)
    att1_row = att1.reshape(1, HEADS * C1).astype(f32)

    grid = (NG,)
    lo_spec = pl.BlockSpec((BT2, UNI), lambda i: (i, 0))
    hi_spec = pl.BlockSpec((BT2, UNI), lambda i: (i + NG, 0))
    w_proj = pl.BlockSpec((HID, UNI), lambda i: (0, 0))
    w_hid = pl.BlockSpec((HID, HID), lambda i: (0, 0))
    vec = pl.BlockSpec((1, HID), lambda i: (0, 0))

    out = pl.pallas_call(
        _gnn_kernel,
        grid=grid,
        in_specs=[
            lo_spec, hi_spec, lo_spec, hi_spec, lo_spec, hi_spec,
            w_proj, w_proj, w_proj,
            w_hid, w_hid, vec,
            w_hid, w_hid, vec,
            pl.BlockSpec((9, HID), lambda i: (0, 0)),
        ],
        out_specs=pl.BlockSpec((2, BT2, HID), lambda i: (0, i, 0)),
        out_shape=jax.ShapeDtypeStruct((2, B // 2, HID), f32),
        scratch_shapes=[
            pltpu.VMEM((3, 2, UNI, 2 * HID), f32),
            pltpu.VMEM((3, 2, UNI, 2 * HID), f32),
            pltpu.VMEM((3, 2 * HID), f32),
            pltpu.VMEM((3, 2 * HID), f32),
            pltpu.VMEM((2 * HID, 2 * HID), f32),
            pltpu.VMEM((2 * HID, 2 * HID), f32),
            pltpu.VMEM((2 * HID, 2 * HID), f32),
            pltpu.VMEM((2 * HID, 2 * HID), f32),
            pltpu.VMEM((4, 2 * HID), f32),
        ],
    )(
        text_features, text_features,
        audio_features, audio_features,
        video_features, video_features,
        W_text, W_audio, W_video,
        Wl1, Wr1, att1_row,
        Wl2, Wr2, att2,
        bvec,
    )
    return out.reshape(B, HID)


# speedup vs baseline: 1.8176x; 1.0098x over previous
import jax
import jax.numpy as jnp
from jax import lax
from jax.experimental import pallas as pl
from jax.experimental.pallas import tpu as pltpu

B = 16384
UNI = 128
HID = 64
HEADS = 4
C1 = HID // HEADS
BT2 = 512
NG = (B // 2) // BT2

_TRANS_RHS = (((1,), (1,)), ((), ()))
_TRANS_LHS = (((0,), (1,)), ((), ()))


def _leaky(x):
    return jnp.maximum(x, 0.2 * x)


def _elu(x):
    return jnp.where(x > 0, x, jnp.exp(x) - 1.0)


def _gnn_kernel(ta_ref, tb_ref, aa_ref, ab_ref, va_ref, vb_ref,
                wt_ref, wa_ref, wv_ref,
                wl1_ref, wr1_ref, att1_ref,
                wl2_ref, wr2_ref, att2_ref,
                bvec_ref, out_ref,
                q_ref, c_ref, w2_ref, c2_ref, ma1_ref, ma2_ref):
    f32 = jnp.float32

    def dg(x, y, dims):
        return lax.dot_general(x, y, dims, preferred_element_type=f32)

    @pl.when(pl.program_id(0) == 0)
    def _prep():
        bv = bvec_ref[...]
        bn = [bv[i:i + 1] for i in range(3)]
        bl1, br1, bias1 = bv[3:4], bv[4:5], bv[5:6]
        bl2, br2, bias2 = bv[6:7], bv[7:8], bv[8:9]
        wl1 = wl1_ref[...]
        wr1 = wr1_ref[...]
        wn = (wt_ref[...], wa_ref[...], wv_ref[...])
        z128 = jnp.zeros((UNI, HID), f32)
        for i in range(3):
            ql = dg(wn[i], wl1, _TRANS_LHS)
            qr = dg(wn[i], wr1, _TRANS_LHS)
            q_ref[i, 0, :, 0 * HID:1 * HID] = ql
            q_ref[i, 0, :, 1 * HID:2 * HID] = z128
            q_ref[i, 0, :, 2 * HID:3 * HID] = qr
            q_ref[i, 0, :, 3 * HID:4 * HID] = z128
            q_ref[i, 1, :, 0 * HID:1 * HID] = z128
            q_ref[i, 1, :, 1 * HID:2 * HID] = ql
            q_ref[i, 1, :, 2 * HID:3 * HID] = z128
            q_ref[i, 1, :, 3 * HID:4 * HID] = qr
            cl = dg(bn[i], wl1, _TRANS_RHS) + bl1 + bias1
            cr = dg(bn[i], wr1, _TRANS_RHS) + br1 - bias1
            c_ref[i:i + 1, 0 * HID:1 * HID] = cl
            c_ref[i:i + 1, 1 * HID:2 * HID] = cl
            c_ref[i:i + 1, 2 * HID:3 * HID] = cr
            c_ref[i:i + 1, 3 * HID:4 * HID] = cr
        rh = lax.broadcasted_iota(jnp.int32, (HID, HID), 0)
        ch = lax.broadcasted_iota(jnp.int32, (HID, HID), 1)
        ident = (rh == ch).astype(f32)
        z64 = jnp.zeros((HID, HID), f32)
        w2lt = dg(ident, wl2_ref[...], _TRANS_RHS) * (1.0 / 3.0)
        w2rt = dg(ident, wr2_ref[...], _TRANS_RHS) * (1.0 / 3.0)
        for half, w2x in ((0, w2lt), (1, w2rt)):
            w2_ref[:HID, (2 * half + 0) * HID:(2 * half + 1) * HID] = w2x
            w2_ref[:HID, (2 * half + 1) * HID:(2 * half + 2) * HID] = z64
            w2_ref[HID:, (2 * half + 0) * HID:(2 * half + 1) * HID] = z64
            w2_ref[HID:, (2 * half + 1) * HID:(2 * half + 2) * HID] = w2x
        cl2 = (bl2 + bias2) * (1.0 / 3.0)
        cr2 = (br2 - bias2) * (1.0 / 3.0)
        c2_ref[0:1, 0 * HID:1 * HID] = cl2
        c2_ref[0:1, 1 * HID:2 * HID] = cl2
        c2_ref[0:1, 2 * HID:3 * HID] = cr2
        c2_ref[0:1, 3 * HID:4 * HID] = cr2
        att1_col = dg(ident, att1_ref[...], _TRANS_RHS)
        ma1 = att1_col * (rh // C1 == ch // C1).astype(f32)
        ma1_ref[:HID, :HID] = ma1
        ma1_ref[:HID, HID:] = z64
        ma1_ref[HID:, :HID] = z64
        ma1_ref[HID:, HID:] = ma1
        att2_col = dg(ident, att2_ref[...], _TRANS_RHS)
        a2 = jnp.broadcast_to(att2_col, (HID, HID)) * 3.0
        ma2_ref[:HID, :HID] = a2
        ma2_ref[:HID, HID:] = z64
        ma2_ref[HID:, :HID] = z64
        ma2_ref[HID:, HID:] = a2

    fa = (ta_ref[...], aa_ref[...], va_ref[...])
    fb = (tb_ref[...], ab_ref[...], vb_ref[...])
    c = c_ref[...]
    c2 = c2_ref[...]

    def mm(x, w):
        return jnp.dot(x, w, preferred_element_type=f32)

    H2 = 2 * HID
    xl, xr = [], []
    for i in range(3):
        z = mm(fa[i], q_ref[i, 0]) + mm(fb[i], q_ref[i, 1]) + c[i:i + 1]
        xl.append(z[:, :H2])
        xr.append(z[:, H2:])

    def gat(xls, xrs, logit_ref):
        outs = []
        for d in range(3):
            a, b = [s for s in range(3) if s != d]
            ea = _leaky(xls[a] + xrs[d])
            eb = _leaky(xls[b] + xrs[d])
            dlog = mm(eb - ea, logit_ref[...])
            sa = 1.0 / (1.0 + jnp.exp(dlog))
            outs.append(xls[b] + sa * (xls[a] - xls[b]))
        return outs

    h = [_elu(o) for o in gat(xl, xr, ma1_ref)]

    yl, yr = [], []
    for i in range(3):
        z = mm(h[i], w2_ref[...]) + c2
        yl.append(z[:, :H2])
        yr.append(z[:, H2:])
    o2 = gat(yl, yr, ma2_ref)
    res = o2[0] + o2[1] + o2[2]
    out_ref[0] = res[:, :HID]
    out_ref[1] = res[:, HID:]


@jax.jit
def kernel(text_features, audio_features, video_features, W_text, b_text,
           W_audio, b_audio, W_video, b_video, Wl1, bl1, Wr1, br1, att1,
           bias1, Wl2, bl2, Wr2, br2, att2, bias2):
    f32 = jnp.float32
    bvec = jnp.stack([b_text, b_audio, b_video, bl1, br1, bias1,
                      bl2, br2, bias2]).astype(f32)
    att1_row = att1.reshape(1, HEADS * C1).astype(f32)

    grid = (NG,)
    lo_spec = pl.BlockSpec((BT2, UNI), lambda i: (i, 0))
    hi_spec = pl.BlockSpec((BT2, UNI), lambda i: (i + NG, 0))
    w_proj = pl.BlockSpec((HID, UNI), lambda i: (0, 0))
    w_hid = pl.BlockSpec((HID, HID), lambda i: (0, 0))
    vec = pl.BlockSpec((1, HID), lambda i: (0, 0))

    out = pl.pallas_call(
        _gnn_kernel,
        grid=grid,
        in_specs=[
            lo_spec, hi_spec, lo_spec, hi_spec, lo_spec, hi_spec,
            w_proj, w_proj, w_proj,
            w_hid, w_hid, vec,
            w_hid, w_hid, vec,
            pl.BlockSpec((9, HID), lambda i: (0, 0)),
        ],
        out_specs=pl.BlockSpec((2, BT2, HID), lambda i: (0, i, 0)),
        out_shape=jax.ShapeDtypeStruct((2, B // 2, HID), f32),
        scratch_shapes=[
            pltpu.VMEM((3, 2, UNI, 4 * HID), f32),
            pltpu.VMEM((3, 4 * HID), f32),
            pltpu.VMEM((2 * HID, 4 * HID), f32),
            pltpu.VMEM((1, 4 * HID), f32),
            pltpu.VMEM((2 * HID, 2 * HID), f32),
            pltpu.VMEM((2 * HID, 2 * HID), f32),
        ],
    )(
        text_features, text_features,
        audio_features, audio_features,
        video_features, video_features,
        W_text, W_audio, W_video,
        Wl1, Wr1, att1_row,
        Wl2, Wr2, att2,
        bvec,
    )
    return out.reshape(B, HID)


# speedup vs baseline: 1.9270x; 1.0602x over previous
import jax
import jax.numpy as jnp
from jax import lax
from jax.experimental import pallas as pl
from jax.experimental.pallas import tpu as pltpu

B = 16384
UNI = 128
HID = 64
HEADS = 4
C1 = HID // HEADS
BT2 = 1024
NG = (B // 2) // BT2

_TRANS_RHS = (((1,), (1,)), ((), ()))
_TRANS_LHS = (((0,), (1,)), ((), ()))


def _leaky(x):
    return jnp.maximum(x, 0.2 * x)


def _elu(x):
    return jnp.where(x > 0, x, jnp.exp(x) - 1.0)


def _gnn_kernel(ta_ref, tb_ref, aa_ref, ab_ref, va_ref, vb_ref,
                wt_ref, wa_ref, wv_ref,
                wl1_ref, wr1_ref, att1_ref,
                wl2_ref, wr2_ref, att2_ref,
                bvec_ref, out_ref,
                q_ref, c_ref, w2_ref, c2_ref, ma1_ref, ma2_ref):
    f32 = jnp.float32

    def dg(x, y, dims):
        return lax.dot_general(x, y, dims, preferred_element_type=f32)

    @pl.when(pl.program_id(0) == 0)
    def _prep():
        bv = bvec_ref[...]
        bn = [bv[i:i + 1] for i in range(3)]
        bl1, br1, bias1 = bv[3:4], bv[4:5], bv[5:6]
        bl2, br2, bias2 = bv[6:7], bv[7:8], bv[8:9]
        wl1 = wl1_ref[...]
        wr1 = wr1_ref[...]
        wn = (wt_ref[...], wa_ref[...], wv_ref[...])
        z128 = jnp.zeros((UNI, HID), f32)
        for i in range(3):
            ql = dg(wn[i], wl1, _TRANS_LHS)
            qr = dg(wn[i], wr1, _TRANS_LHS)
            q_ref[i, 0, :, 0 * HID:1 * HID] = ql
            q_ref[i, 0, :, 1 * HID:2 * HID] = z128
            q_ref[i, 0, :, 2 * HID:3 * HID] = qr
            q_ref[i, 0, :, 3 * HID:4 * HID] = z128
            q_ref[i, 1, :, 0 * HID:1 * HID] = z128
            q_ref[i, 1, :, 1 * HID:2 * HID] = ql
            q_ref[i, 1, :, 2 * HID:3 * HID] = z128
            q_ref[i, 1, :, 3 * HID:4 * HID] = qr
            cl = dg(bn[i], wl1, _TRANS_RHS) + bl1 + bias1
            cr = dg(bn[i], wr1, _TRANS_RHS) + br1 - bias1
            c_ref[i:i + 1, 0 * HID:1 * HID] = cl
            c_ref[i:i + 1, 1 * HID:2 * HID] = cl
            c_ref[i:i + 1, 2 * HID:3 * HID] = cr
            c_ref[i:i + 1, 3 * HID:4 * HID] = cr
        rh = lax.broadcasted_iota(jnp.int32, (HID, HID), 0)
        ch = lax.broadcasted_iota(jnp.int32, (HID, HID), 1)
        ident = (rh == ch).astype(f32)
        z64 = jnp.zeros((HID, HID), f32)
        w2lt = dg(ident, wl2_ref[...], _TRANS_RHS) * (1.0 / 3.0)
        w2rt = dg(ident, wr2_ref[...], _TRANS_RHS) * (1.0 / 3.0)
        for half, w2x in ((0, w2lt), (1, w2rt)):
            w2_ref[:HID, (2 * half + 0) * HID:(2 * half + 1) * HID] = w2x
            w2_ref[:HID, (2 * half + 1) * HID:(2 * half + 2) * HID] = z64
            w2_ref[HID:, (2 * half + 0) * HID:(2 * half + 1) * HID] = z64
            w2_ref[HID:, (2 * half + 1) * HID:(2 * half + 2) * HID] = w2x
        cl2 = (bl2 + bias2) * (1.0 / 3.0)
        cr2 = (br2 - bias2) * (1.0 / 3.0)
        c2_ref[0:1, 0 * HID:1 * HID] = cl2
        c2_ref[0:1, 1 * HID:2 * HID] = cl2
        c2_ref[0:1, 2 * HID:3 * HID] = cr2
        c2_ref[0:1, 3 * HID:4 * HID] = cr2
        att1_col = dg(ident, att1_ref[...], _TRANS_RHS)
        ma1 = att1_col * (rh // C1 == ch // C1).astype(f32)
        ma1_ref[:HID, :HID] = ma1
        ma1_ref[:HID, HID:] = z64
        ma1_ref[HID:, :HID] = z64
        ma1_ref[HID:, HID:] = ma1
        att2_col = dg(ident, att2_ref[...], _TRANS_RHS)
        a2 = jnp.broadcast_to(att2_col, (HID, HID)) * 3.0
        ma2_ref[:HID, :HID] = a2
        ma2_ref[:HID, HID:] = z64
        ma2_ref[HID:, :HID] = z64
        ma2_ref[HID:, HID:] = a2

    fa = (ta_ref[...], aa_ref[...], va_ref[...])
    fb = (tb_ref[...], ab_ref[...], vb_ref[...])
    c = c_ref[...]
    c2 = c2_ref[...]

    def mm(x, w):
        return jnp.dot(x, w, preferred_element_type=f32)

    H2 = 2 * HID
    xl, xr = [], []
    for i in range(3):
        z = mm(fa[i], q_ref[i, 0]) + mm(fb[i], q_ref[i, 1]) + c[i:i + 1]
        xl.append(z[:, :H2])
        xr.append(z[:, H2:])

    def gat(xls, xrs, logit_ref):
        outs = []
        for d in range(3):
            a, b = [s for s in range(3) if s != d]
            ea = _leaky(xls[a] + xrs[d])
            eb = _leaky(xls[b] + xrs[d])
            dlog = mm(eb - ea, logit_ref[...])
            sa = 1.0 / (1.0 + jnp.exp(dlog))
            outs.append(xls[b] + sa * (xls[a] - xls[b]))
        return outs

    h = [_elu(o) for o in gat(xl, xr, ma1_ref)]

    yl, yr = [], []
    for i in range(3):
        z = mm(h[i], w2_ref[...]) + c2
        yl.append(z[:, :H2])
        yr.append(z[:, H2:])
    o2 = gat(yl, yr, ma2_ref)
    res = o2[0] + o2[1] + o2[2]
    out_ref[0] = res[:, :HID]
    out_ref[1] = res[:, HID:]


@jax.jit
def kernel(text_features, audio_features, video_features, W_text, b_text,
           W_audio, b_audio, W_video, b_video, Wl1, bl1, Wr1, br1, att1,
           bias1, Wl2, bl2, Wr2, br2, att2, bias2):
    f32 = jnp.float32
    bvec = jnp.stack([b_text, b_audio, b_video, bl1, br1, bias1,
                      bl2, br2, bias2]).astype(f32)
    att1_row = att1.reshape(1, HEADS * C1).astype(f32)

    grid = (NG,)
    lo_spec = pl.BlockSpec((BT2, UNI), lambda i: (i, 0))
    hi_spec = pl.BlockSpec((BT2, UNI), lambda i: (i + NG, 0))
    w_proj = pl.BlockSpec((HID, UNI), lambda i: (0, 0))
    w_hid = pl.BlockSpec((HID, HID), lambda i: (0, 0))
    vec = pl.BlockSpec((1, HID), lambda i: (0, 0))

    out = pl.pallas_call(
        _gnn_kernel,
        grid=grid,
        in_specs=[
            lo_spec, hi_spec, lo_spec, hi_spec, lo_spec, hi_spec,
            w_proj, w_proj, w_proj,
            w_hid, w_hid, vec,
            w_hid, w_hid, vec,
            pl.BlockSpec((9, HID), lambda i: (0, 0)),
        ],
        out_specs=pl.BlockSpec((2, BT2, HID), lambda i: (0, i, 0)),
        out_shape=jax.ShapeDtypeStruct((2, B // 2, HID), f32),
        scratch_shapes=[
            pltpu.VMEM((3, 2, UNI, 4 * HID), f32),
            pltpu.VMEM((3, 4 * HID), f32),
            pltpu.VMEM((2 * HID, 4 * HID), f32),
            pltpu.VMEM((1, 4 * HID), f32),
            pltpu.VMEM((2 * HID, 2 * HID), f32),
            pltpu.VMEM((2 * HID, 2 * HID), f32),
        ],
    )(
        text_features, text_features,
        audio_features, audio_features,
        video_features, video_features,
        W_text, W_audio, W_video,
        Wl1, Wr1, att1_row,
        Wl2, Wr2, att2,
        bvec,
    )
    return out.reshape(B, HID)
